# baseline (speedup 1.0000x reference)
import functools

import jax
import jax.numpy as jnp
from jax import lax
from jax.experimental import pallas as pl
from jax.experimental.pallas import tpu as pltpu
from jax.experimental.pallas import tpu_sc as plsc

N = 10000
E = 320000
D = 128
DEPTH = 10
TELEPORT = 0.5

NC, NS = 2, 16
NW = NC * NS
C = 128
C2 = 256
NCH = 40
EPW = C2 * NCH
EPAD = EPW * NW
NPAD = 10112
ZCH = 79

_mesh = plsc.VectorSubcoreMesh(core_axis_name="c", subcore_axis_name="s")
_sc_params = pltpu.CompilerParams(use_tc_tiling_on_sc=False)


@functools.partial(
    pl.kernel,
    out_type=jax.ShapeDtypeStruct((NC, NPAD, D), jnp.float32),
    mesh=_mesh,
    scratch_types=[
        pltpu.VMEM((2 * C2,), jnp.int32),
        pltpu.VMEM((C2, D), jnp.float32),
        pltpu.VMEM((C, D), jnp.float32),
        pltpu.VMEM_SHARED((NPAD, D), jnp.float32),
        pltpu.SemaphoreType.DMA,
    ],
    compiler_params=_sc_params,
)
def _edge_step(out_hbm, il_hbm, acc_hbm, idx2, rows, buf, acc_sh, sg):
    cid = lax.axis_index("c")
    sid = lax.axis_index("s")
    wid = cid * NS + sid

    zero16 = jnp.zeros((16,), jnp.float32)

    def _zb(t, _):
        buf[t // 8, pl.ds((t % 8) * 16, 16)] = zero16
        return 0
    lax.fori_loop(0, C * (D // 16), _zb, 0)

    rows_per_tile = NPAD // NS

    def _za(k, _):
        pltpu.sync_copy(buf.at[pl.ds(0, ZCH)],
                        acc_sh.at[pl.ds(sid * rows_per_tile + k * ZCH, ZCH)])
        return 0
    lax.fori_loop(0, rows_per_tile // ZCH, _za, 0)
    plsc.subcore_barrier()

    def _edge(j, _):
        base = (wid * NCH + j) * 2 * C2
        pltpu.sync_copy(il_hbm.at[pl.ds(base, 2 * C2)], idx2)
        pltpu.async_copy(out_hbm.at[idx2.at[pl.ds(0, C2)]], rows, sg).wait()
        pltpu.sync_copy(rows, acc_sh.at[idx2.at[pl.ds(C2, C2)]], add=True)
        return 0
    lax.fori_loop(0, NCH, _edge, 0)
    plsc.subcore_barrier()

    def _wb(k, _):
        r0 = sid * rows_per_tile + k * ZCH
        pltpu.sync_copy(acc_sh.at[pl.ds(r0, ZCH)], buf.at[pl.ds(0, ZCH)])
        pltpu.sync_copy(buf.at[pl.ds(0, ZCH)], acc_hbm.at[cid, pl.ds(r0, ZCH)])
        return 0
    lax.fori_loop(0, rows_per_tile // ZCH, _wb, 0)


@functools.partial(
    pl.kernel,
    out_type=jax.ShapeDtypeStruct((NC, NPAD, 16), jnp.float32),
    mesh=_mesh,
    scratch_types=[
        pltpu.VMEM((C,), jnp.int32),
        pltpu.VMEM((C, 16), jnp.float32),
        pltpu.VMEM((C, 16), jnp.float32),
        pltpu.VMEM((NPAD // NS, 16), jnp.float32),
        pltpu.VMEM_SHARED((NPAD, 16), jnp.float32),
    ],
    compiler_params=_sc_params,
)
def _deg_step(dst_hbm, deg_hbm, dst_c, ones_b, zero_b, dbuf, deg_sh):
    cid = lax.axis_index("c")
    sid = lax.axis_index("s")
    wid = cid * NS + sid

    one16 = jnp.ones((16,), jnp.float32)
    zero16 = jnp.zeros((16,), jnp.float32)

    def _fill(i, _):
        ones_b[i, :] = one16
        zero_b[i, :] = zero16
        return 0
    lax.fori_loop(0, C, _fill, 0)

    rows_per_tile = NPAD // NS

    def _za(k, _):
        pltpu.sync_copy(zero_b.at[pl.ds(0, ZCH)],
                        deg_sh.at[pl.ds(sid * rows_per_tile + k * ZCH, ZCH)])
        return 0
    lax.fori_loop(0, rows_per_tile // ZCH, _za, 0)
    plsc.subcore_barrier()

    def _edge(j, _):
        pltpu.sync_copy(dst_hbm.at[pl.ds(wid * EPW + j * C, C)], dst_c)
        pltpu.sync_copy(ones_b, deg_sh.at[dst_c], add=True)
        return 0
    lax.fori_loop(0, EPW // C, _edge, 0)
    plsc.subcore_barrier()

    r0 = sid * rows_per_tile
    pltpu.sync_copy(deg_sh.at[pl.ds(r0, rows_per_tile)], dbuf)
    pltpu.sync_copy(dbuf, deg_hbm.at[cid, pl.ds(r0, rows_per_tile)])


def _emb_body(x_ref, w_ref, b_ref, o_ref):
    x = x_ref[...]
    w = w_ref[...]
    o_ref[...] = lax.dot_general(
        x, w, (((1,), (1,)), ((), ())),
        preferred_element_type=jnp.float32) + b_ref[...]


_BLK = 1000


def _emb_call(data, W, b2):
    return pl.pallas_call(
        _emb_body,
        grid=(N // _BLK,),
        in_specs=[
            pl.BlockSpec((_BLK, D), lambda i: (i, 0)),
            pl.BlockSpec((D, D), lambda i: (0, 0)),
            pl.BlockSpec((1, D), lambda i: (0, 0)),
        ],
        out_specs=pl.BlockSpec((_BLK, D), lambda i: (i, 0)),
        out_shape=jax.ShapeDtypeStruct((N, D), jnp.float32),
    )(data, W, b2)


def _upd_body(relu, out_ref, a0_ref, a1_ref, emb_ref, d0_ref, d1_ref, o_ref):
    deg = d0_ref[:, 0:1] + d1_ref[:, 0:1]
    pos = deg > 0.5
    invd = jnp.where(pos, (1.0 - TELEPORT) / jnp.where(pos, deg, 1.0), 0.0)
    a = jnp.where(pos, 1.0 - TELEPORT, 0.0)
    acc = a0_ref[...] + a1_ref[...]
    newo = a * out_ref[...] - invd * acc + TELEPORT * emb_ref[...]
    o_ref[...] = jnp.maximum(newo, 0.0) if relu else newo


def _upd_call(relu, out, acc0, acc1, emb, d0, d1):
    return pl.pallas_call(
        functools.partial(_upd_body, relu),
        grid=(N // _BLK,),
        in_specs=[
            pl.BlockSpec((_BLK, D), lambda i: (i, 0)),
            pl.BlockSpec((_BLK, D), lambda i: (i, 0)),
            pl.BlockSpec((_BLK, D), lambda i: (i, 0)),
            pl.BlockSpec((_BLK, D), lambda i: (i, 0)),
            pl.BlockSpec((_BLK, 16), lambda i: (i, 0)),
            pl.BlockSpec((_BLK, 16), lambda i: (i, 0)),
        ],
        out_specs=pl.BlockSpec((_BLK, D), lambda i: (i, 0)),
        out_shape=jax.ShapeDtypeStruct((N, D), jnp.float32),
    )(out, acc0, acc1, emb, d0, d1)


def kernel(data, edge_index, W, b):
    src = edge_index[0].astype(jnp.int32)
    dst = edge_index[1].astype(jnp.int32)
    pad = EPAD - E
    srcp = jnp.concatenate([src, jnp.zeros((pad,), jnp.int32)])
    dstp = jnp.concatenate([dst, jnp.full((pad,), N, jnp.int32)])
    ilp = jnp.stack([srcp.reshape(NW * NCH, C2),
                     dstp.reshape(NW * NCH, C2)], axis=1).reshape(-1)

    emb = _emb_call(data, W, b.reshape(1, D))
    deg2 = _deg_step(dstp)
    d0, d1 = deg2[0], deg2[1]

    out = emb
    for t in range(DEPTH):
        acc2 = _edge_step(out, ilp)
        out = _upd_call(t == DEPTH - 1, out, acc2[0], acc2[1], emb, d0, d1)
    return out


# speedup vs baseline: 1.0451x; 1.0451x over previous
import functools

import jax
import jax.numpy as jnp
from jax import lax
from jax.experimental import pallas as pl
from jax.experimental.pallas import tpu as pltpu
from jax.experimental.pallas import tpu_sc as plsc

N = 10000
E = 320000
D = 128
DEPTH = 10
TELEPORT = 0.5

NC, NS = 2, 16
NW = NC * NS
C = 128
NCH = 80
EPW = C * NCH
EPAD = EPW * NW
NPAD = 10112
ZCH = 79

_mesh = plsc.VectorSubcoreMesh(core_axis_name="c", subcore_axis_name="s")
_sc_params = pltpu.CompilerParams(use_tc_tiling_on_sc=False)


@functools.partial(
    pl.kernel,
    out_type=jax.ShapeDtypeStruct((NC, NPAD, D), jnp.float32),
    mesh=_mesh,
    scratch_types=[
        pltpu.VMEM((C,), jnp.int32),
        pltpu.VMEM((C,), jnp.int32),
        pltpu.VMEM((C,), jnp.int32),
        pltpu.VMEM((C,), jnp.int32),
        pltpu.VMEM((C, D), jnp.float32),
        pltpu.VMEM((C, D), jnp.float32),
        pltpu.VMEM_SHARED((NPAD, D), jnp.float32),
        pltpu.SemaphoreType.DMA,
        pltpu.SemaphoreType.DMA,
        pltpu.SemaphoreType.DMA,
    ],
    compiler_params=_sc_params,
)
def _edge_step(out_hbm, src_hbm, dst_hbm, acc_hbm,
               src_a, dst_a, src_b, dst_b, rows, buf, acc_sh, sg, sia, sib):
    cid = lax.axis_index("c")
    sid = lax.axis_index("s")
    wid = cid * NS + sid

    zero16 = jnp.zeros((16,), jnp.float32)

    def _zb(t, _):
        buf[t // 8, pl.ds((t % 8) * 16, 16)] = zero16
        return 0
    lax.fori_loop(0, C * (D // 16), _zb, 0)

    rows_per_tile = NPAD // NS

    def _za(k, _):
        pltpu.sync_copy(buf.at[pl.ds(0, ZCH)],
                        acc_sh.at[pl.ds(sid * rows_per_tile + k * ZCH, ZCH)])
        return 0
    lax.fori_loop(0, rows_per_tile // ZCH, _za, 0)
    plsc.subcore_barrier()

    pairs = ((src_a, dst_a, sia), (src_b, dst_b, sib))

    def issue_idx(j, p):
        s, d, si = pairs[p]
        base = wid * EPW + j * C
        pltpu.async_copy(src_hbm.at[pl.ds(base, C)], s, si)
        pltpu.async_copy(dst_hbm.at[pl.ds(base, C)], d, si)

    def wait_idx(p):
        s, d, si = pairs[p]
        pltpu.make_async_copy(src_hbm.at[pl.ds(0, C)], s, si).wait()
        pltpu.make_async_copy(dst_hbm.at[pl.ds(0, C)], d, si).wait()

    issue_idx(0, 0)

    def _edge(g, _):
        for k in (0, 1):
            j = 2 * g + k
            s, d, _si = pairs[k]
            wait_idx(k)
            issue_idx(j + 1, 1 - k)
            pltpu.async_copy(out_hbm.at[s], rows, sg).wait()
            pltpu.sync_copy(rows, acc_sh.at[d], add=True)
        return 0
    lax.fori_loop(0, NCH // 2 - 1, _edge, 0)
    wait_idx(0)
    issue_idx(NCH - 1, 1)
    pltpu.async_copy(out_hbm.at[src_a], rows, sg).wait()
    pltpu.sync_copy(rows, acc_sh.at[dst_a], add=True)
    wait_idx(1)
    pltpu.async_copy(out_hbm.at[src_b], rows, sg).wait()
    pltpu.sync_copy(rows, acc_sh.at[dst_b], add=True)
    plsc.subcore_barrier()

    def _wb(k, _):
        r0 = sid * rows_per_tile + k * ZCH
        pltpu.sync_copy(acc_sh.at[pl.ds(r0, ZCH)], buf.at[pl.ds(0, ZCH)])
        pltpu.sync_copy(buf.at[pl.ds(0, ZCH)], acc_hbm.at[cid, pl.ds(r0, ZCH)])
        return 0
    lax.fori_loop(0, rows_per_tile // ZCH, _wb, 0)


@functools.partial(
    pl.kernel,
    out_type=jax.ShapeDtypeStruct((NC, NPAD, 16), jnp.float32),
    mesh=_mesh,
    scratch_types=[
        pltpu.VMEM((C,), jnp.int32),
        pltpu.VMEM((C, 16), jnp.float32),
        pltpu.VMEM((C, 16), jnp.float32),
        pltpu.VMEM((NPAD // NS, 16), jnp.float32),
        pltpu.VMEM_SHARED((NPAD, 16), jnp.float32),
    ],
    compiler_params=_sc_params,
)
def _deg_step(dst_hbm, deg_hbm, dst_c, ones_b, zero_b, dbuf, deg_sh):
    cid = lax.axis_index("c")
    sid = lax.axis_index("s")
    wid = cid * NS + sid

    one16 = jnp.ones((16,), jnp.float32)
    zero16 = jnp.zeros((16,), jnp.float32)

    def _fill(i, _):
        ones_b[i, :] = one16
        zero_b[i, :] = zero16
        return 0
    lax.fori_loop(0, C, _fill, 0)

    rows_per_tile = NPAD // NS

    def _za(k, _):
        pltpu.sync_copy(zero_b.at[pl.ds(0, ZCH)],
                        deg_sh.at[pl.ds(sid * rows_per_tile + k * ZCH, ZCH)])
        return 0
    lax.fori_loop(0, rows_per_tile // ZCH, _za, 0)
    plsc.subcore_barrier()

    def _edge(j, _):
        pltpu.sync_copy(dst_hbm.at[pl.ds(wid * EPW + j * C, C)], dst_c)
        pltpu.sync_copy(ones_b, deg_sh.at[dst_c], add=True)
        return 0
    lax.fori_loop(0, EPW // C, _edge, 0)
    plsc.subcore_barrier()

    r0 = sid * rows_per_tile
    pltpu.sync_copy(deg_sh.at[pl.ds(r0, rows_per_tile)], dbuf)
    pltpu.sync_copy(dbuf, deg_hbm.at[cid, pl.ds(r0, rows_per_tile)])


def _emb_body(x_ref, w_ref, b_ref, o_ref):
    x = x_ref[...]
    w = w_ref[...]
    o_ref[...] = lax.dot_general(
        x, w, (((1,), (1,)), ((), ())),
        preferred_element_type=jnp.float32) + b_ref[...]


_BLK = 1000


def _emb_call(data, W, b2):
    return pl.pallas_call(
        _emb_body,
        grid=(N // _BLK,),
        in_specs=[
            pl.BlockSpec((_BLK, D), lambda i: (i, 0)),
            pl.BlockSpec((D, D), lambda i: (0, 0)),
            pl.BlockSpec((1, D), lambda i: (0, 0)),
        ],
        out_specs=pl.BlockSpec((_BLK, D), lambda i: (i, 0)),
        out_shape=jax.ShapeDtypeStruct((N, D), jnp.float32),
    )(data, W, b2)


def _upd_body(relu, out_ref, a0_ref, a1_ref, emb_ref, d0_ref, d1_ref, o_ref):
    deg = d0_ref[:, 0:1] + d1_ref[:, 0:1]
    pos = deg > 0.5
    invd = jnp.where(pos, (1.0 - TELEPORT) / jnp.where(pos, deg, 1.0), 0.0)
    a = jnp.where(pos, 1.0 - TELEPORT, 0.0)
    acc = a0_ref[...] + a1_ref[...]
    newo = a * out_ref[...] - invd * acc + TELEPORT * emb_ref[...]
    o_ref[...] = jnp.maximum(newo, 0.0) if relu else newo


def _upd_call(relu, out, acc0, acc1, emb, d0, d1):
    return pl.pallas_call(
        functools.partial(_upd_body, relu),
        grid=(N // _BLK,),
        in_specs=[
            pl.BlockSpec((_BLK, D), lambda i: (i, 0)),
            pl.BlockSpec((_BLK, D), lambda i: (i, 0)),
            pl.BlockSpec((_BLK, D), lambda i: (i, 0)),
            pl.BlockSpec((_BLK, D), lambda i: (i, 0)),
            pl.BlockSpec((_BLK, 16), lambda i: (i, 0)),
            pl.BlockSpec((_BLK, 16), lambda i: (i, 0)),
        ],
        out_specs=pl.BlockSpec((_BLK, D), lambda i: (i, 0)),
        out_shape=jax.ShapeDtypeStruct((N, D), jnp.float32),
    )(out, acc0, acc1, emb, d0, d1)


def kernel(data, edge_index, W, b):
    src = edge_index[0].astype(jnp.int32)
    dst = edge_index[1].astype(jnp.int32)
    pad = EPAD - E
    srcp = jnp.concatenate([src, jnp.zeros((pad,), jnp.int32)])
    dstp = jnp.concatenate([dst, jnp.full((pad,), N, jnp.int32)])

    emb = _emb_call(data, W, b.reshape(1, D))
    deg2 = _deg_step(dstp)
    d0, d1 = deg2[0], deg2[1]

    out = emb
    for t in range(DEPTH):
        acc2 = _edge_step(out, srcp, dstp)
        out = _upd_call(t == DEPTH - 1, out, acc2[0], acc2[1], emb, d0, d1)
    return out


# speedup vs baseline: 1.3436x; 1.2856x over previous
import functools

import jax
import jax.numpy as jnp
from jax import lax
from jax.experimental import pallas as pl
from jax.experimental.pallas import tpu as pltpu
from jax.experimental.pallas import tpu_sc as plsc

N = 10000
E = 320000
D = 128
DEPTH = 10
TELEPORT = 0.5

NC, NS = 2, 16
NW = NC * NS
C = 128
NCH = 79
EPW = C * NCH
EPAD = EPW * NW
NPAD = 10240

_mesh = plsc.VectorSubcoreMesh(core_axis_name="c", subcore_axis_name="s")
_sc_params = pltpu.CompilerParams(use_tc_tiling_on_sc=False)


@functools.partial(
    pl.kernel,
    out_type=jax.ShapeDtypeStruct((NC, NPAD, D), jnp.float32),
    mesh=_mesh,
    scratch_types=[
        pltpu.VMEM((C,), jnp.int32),
        pltpu.VMEM((C,), jnp.int32),
        pltpu.VMEM((C, D), jnp.float32),
        pltpu.VMEM((C, D), jnp.float32),
        pltpu.VMEM_SHARED((NPAD, D), jnp.float32),
        pltpu.SemaphoreType.DMA,
    ],
    compiler_params=_sc_params,
)
def _edge_step(out_hbm, src_hbm, dst_hbm, acc_hbm,
               src_c, dst_c, rows, buf, acc_sh, sem):
    cid = lax.axis_index("c")
    sid = lax.axis_index("s")
    wid = cid * NS + sid

    zero16 = jnp.zeros((16,), jnp.float32)

    def _zb(t, _):
        buf[t // 8, pl.ds((t % 8) * 16, 16)] = zero16
        return 0
    lax.fori_loop(0, C * (D // 16), _zb, 0)

    rows_per_tile = NPAD // NS

    def _za(k, _):
        pltpu.sync_copy(buf, acc_sh.at[pl.ds(sid * rows_per_tile + k * C, C)])
        return 0
    lax.fori_loop(0, rows_per_tile // C, _za, 0)
    plsc.subcore_barrier()

    def _edge(j, _):
        base = wid * EPW + j * C
        pltpu.sync_copy(src_hbm.at[pl.ds(base, C)], src_c)
        pltpu.sync_copy(dst_hbm.at[pl.ds(base, C)], dst_c)
        pltpu.async_copy(out_hbm.at[src_c], rows, sem).wait()
        pltpu.sync_copy(rows, acc_sh.at[dst_c], add=True)
        return 0
    lax.fori_loop(0, NCH, _edge, 0)
    plsc.subcore_barrier()

    def _wb(k, _):
        r0 = sid * rows_per_tile + k * C
        pltpu.sync_copy(acc_sh.at[pl.ds(r0, C)], buf)
        pltpu.sync_copy(buf, acc_hbm.at[cid, pl.ds(r0, C)])
        return 0
    lax.fori_loop(0, rows_per_tile // C, _wb, 0)


@functools.partial(
    pl.kernel,
    out_type=jax.ShapeDtypeStruct((NC, NPAD, 16), jnp.float32),
    mesh=_mesh,
    scratch_types=[
        pltpu.VMEM((C,), jnp.int32),
        pltpu.VMEM((C, 16), jnp.float32),
        pltpu.VMEM((C, 16), jnp.float32),
        pltpu.VMEM((NPAD // NS, 16), jnp.float32),
        pltpu.VMEM_SHARED((NPAD, 16), jnp.float32),
    ],
    compiler_params=_sc_params,
)
def _deg_step(dst_hbm, deg_hbm, dst_c, ones_b, zero_b, dbuf, deg_sh):
    cid = lax.axis_index("c")
    sid = lax.axis_index("s")
    wid = cid * NS + sid

    one16 = jnp.ones((16,), jnp.float32)
    zero16 = jnp.zeros((16,), jnp.float32)

    def _fill(i, _):
        ones_b[i, :] = one16
        zero_b[i, :] = zero16
        return 0
    lax.fori_loop(0, C, _fill, 0)

    rows_per_tile = NPAD // NS

    def _za(k, _):
        pltpu.sync_copy(zero_b, deg_sh.at[pl.ds(sid * rows_per_tile + k * C, C)])
        return 0
    lax.fori_loop(0, rows_per_tile // C, _za, 0)
    plsc.subcore_barrier()

    def _edge(j, _):
        pltpu.sync_copy(dst_hbm.at[pl.ds(wid * EPW + j * C, C)], dst_c)
        pltpu.sync_copy(ones_b, deg_sh.at[dst_c], add=True)
        return 0
    lax.fori_loop(0, NCH, _edge, 0)
    plsc.subcore_barrier()

    r0 = sid * rows_per_tile
    pltpu.sync_copy(deg_sh.at[pl.ds(r0, rows_per_tile)], dbuf)
    pltpu.sync_copy(dbuf, deg_hbm.at[cid, pl.ds(r0, rows_per_tile)])


def _emb_body(x_ref, w_ref, b_ref, o_ref):
    x = x_ref[...]
    w = w_ref[...]
    o_ref[...] = lax.dot_general(
        x, w, (((1,), (1,)), ((), ())),
        preferred_element_type=jnp.float32) + b_ref[...]


_BLK = 1000


def _emb_call(data, W, b2):
    return pl.pallas_call(
        _emb_body,
        grid=(N // _BLK,),
        in_specs=[
            pl.BlockSpec((_BLK, D), lambda i: (i, 0)),
            pl.BlockSpec((D, D), lambda i: (0, 0)),
            pl.BlockSpec((1, D), lambda i: (0, 0)),
        ],
        out_specs=pl.BlockSpec((_BLK, D), lambda i: (i, 0)),
        out_shape=jax.ShapeDtypeStruct((N, D), jnp.float32),
    )(data, W, b2)


def _upd_body(relu, out_ref, a0_ref, a1_ref, emb_ref, d0_ref, d1_ref, o_ref):
    deg = d0_ref[:, 0:1] + d1_ref[:, 0:1]
    pos = deg > 0.5
    invd = jnp.where(pos, (1.0 - TELEPORT) / jnp.where(pos, deg, 1.0), 0.0)
    a = jnp.where(pos, 1.0 - TELEPORT, 0.0)
    acc = a0_ref[...] + a1_ref[...]
    newo = a * out_ref[...] - invd * acc + TELEPORT * emb_ref[...]
    o_ref[...] = jnp.maximum(newo, 0.0) if relu else newo


def _upd_call(relu, out, acc0, acc1, emb, d0, d1):
    return pl.pallas_call(
        functools.partial(_upd_body, relu),
        grid=(N // _BLK,),
        in_specs=[
            pl.BlockSpec((_BLK, D), lambda i: (i, 0)),
            pl.BlockSpec((_BLK, D), lambda i: (i, 0)),
            pl.BlockSpec((_BLK, D), lambda i: (i, 0)),
            pl.BlockSpec((_BLK, D), lambda i: (i, 0)),
            pl.BlockSpec((_BLK, 16), lambda i: (i, 0)),
            pl.BlockSpec((_BLK, 16), lambda i: (i, 0)),
        ],
        out_specs=pl.BlockSpec((_BLK, D), lambda i: (i, 0)),
        out_shape=jax.ShapeDtypeStruct((N, D), jnp.float32),
    )(out, acc0, acc1, emb, d0, d1)


def kernel(data, edge_index, W, b):
    src = edge_index[0].astype(jnp.int32)
    dst = edge_index[1].astype(jnp.int32)
    pad = EPAD - E
    srcp = jnp.concatenate([src, jnp.zeros((pad,), jnp.int32)])
    dstp = jnp.concatenate([dst, jnp.full((pad,), N, jnp.int32)])

    emb = _emb_call(data, W, b.reshape(1, D))
    deg2 = _deg_step(dstp)
    d0, d1 = deg2[0], deg2[1]

    out = emb
    for t in range(DEPTH):
        acc2 = _edge_step(out, srcp, dstp)
        out = _upd_call(t == DEPTH - 1, out, acc2[0], acc2[1], emb, d0, d1)
    return out


# speedup vs baseline: 1.4468x; 1.0769x over previous
import functools

import jax
import jax.numpy as jnp
from jax import lax
from jax.experimental import pallas as pl
from jax.experimental.pallas import tpu as pltpu
from jax.experimental.pallas import tpu_sc as plsc

N = 10000
E = 320000
D = 128
DEPTH = 10
TELEPORT = 0.5

NC, NS = 2, 16
NW = NC * NS
C = 128
NCH = 79
EPW = C * NCH
EPAD = EPW * NW
NPAD = 10240

_mesh = plsc.VectorSubcoreMesh(core_axis_name="c", subcore_axis_name="s")
_sc_params = pltpu.CompilerParams(use_tc_tiling_on_sc=False)


@functools.partial(
    pl.kernel,
    out_type=jax.ShapeDtypeStruct((NC, NPAD, D), jnp.float32),
    mesh=_mesh,
    scratch_types=[
        pltpu.VMEM((C,), jnp.int32),
        pltpu.VMEM((C,), jnp.int32),
        pltpu.VMEM((C, D), jnp.float32),
        pltpu.VMEM((C, D), jnp.float32),
        pltpu.VMEM_SHARED((NPAD, D), jnp.float32),
        pltpu.SemaphoreType.DMA,
    ],
    compiler_params=_sc_params,
)
def _edge_step(out_hbm, src_hbm, dst_hbm, zer_hbm, acc_hbm,
               src_c, dst_c, rows, buf, acc_sh, sem):
    cid = lax.axis_index("c")
    sid = lax.axis_index("s")
    wid = cid * NS + sid

    pltpu.sync_copy(zer_hbm, buf)

    rows_per_tile = NPAD // NS

    def _za(k, _):
        pltpu.sync_copy(buf, acc_sh.at[pl.ds(sid * rows_per_tile + k * C, C)])
        return 0
    lax.fori_loop(0, rows_per_tile // C, _za, 0)
    plsc.subcore_barrier()

    def _edge(j, _):
        base = wid * EPW + j * C
        d1 = pltpu.async_copy(src_hbm.at[pl.ds(base, C)], src_c, sem)
        d2 = pltpu.async_copy(dst_hbm.at[pl.ds(base, C)], dst_c, sem)
        d1.wait()
        d2.wait()
        pltpu.async_copy(out_hbm.at[src_c], rows, sem).wait()
        pltpu.sync_copy(rows, acc_sh.at[dst_c], add=True)
        return 0
    lax.fori_loop(0, NCH, _edge, 0)
    plsc.subcore_barrier()

    def _wb(k, _):
        r0 = sid * rows_per_tile + k * C
        pltpu.sync_copy(acc_sh.at[pl.ds(r0, C)], buf)
        pltpu.sync_copy(buf, acc_hbm.at[cid, pl.ds(r0, C)])
        return 0
    lax.fori_loop(0, rows_per_tile // C, _wb, 0)


@functools.partial(
    pl.kernel,
    out_type=jax.ShapeDtypeStruct((NC, NPAD, 16), jnp.float32),
    mesh=_mesh,
    scratch_types=[
        pltpu.VMEM((C,), jnp.int32),
        pltpu.VMEM((C, 16), jnp.float32),
        pltpu.VMEM((C, 16), jnp.float32),
        pltpu.VMEM((NPAD // NS, 16), jnp.float32),
        pltpu.VMEM_SHARED((NPAD, 16), jnp.float32),
    ],
    compiler_params=_sc_params,
)
def _deg_step(dst_hbm, deg_hbm, dst_c, ones_b, zero_b, dbuf, deg_sh):
    cid = lax.axis_index("c")
    sid = lax.axis_index("s")
    wid = cid * NS + sid

    one16 = jnp.ones((16,), jnp.float32)
    zero16 = jnp.zeros((16,), jnp.float32)

    def _fill(i, _):
        ones_b[i, :] = one16
        zero_b[i, :] = zero16
        return 0
    lax.fori_loop(0, C, _fill, 0)

    rows_per_tile = NPAD // NS

    def _za(k, _):
        pltpu.sync_copy(zero_b, deg_sh.at[pl.ds(sid * rows_per_tile + k * C, C)])
        return 0
    lax.fori_loop(0, rows_per_tile // C, _za, 0)
    plsc.subcore_barrier()

    def _edge(j, _):
        pltpu.sync_copy(dst_hbm.at[pl.ds(wid * EPW + j * C, C)], dst_c)
        pltpu.sync_copy(ones_b, deg_sh.at[dst_c], add=True)
        return 0
    lax.fori_loop(0, NCH, _edge, 0)
    plsc.subcore_barrier()

    r0 = sid * rows_per_tile
    pltpu.sync_copy(deg_sh.at[pl.ds(r0, rows_per_tile)], dbuf)
    pltpu.sync_copy(dbuf, deg_hbm.at[cid, pl.ds(r0, rows_per_tile)])


def _emb_body(x_ref, w_ref, b_ref, o_ref):
    x = x_ref[...]
    w = w_ref[...]
    o_ref[...] = lax.dot_general(
        x, w, (((1,), (1,)), ((), ())),
        preferred_element_type=jnp.float32) + b_ref[...]


_BLK = 1000


def _emb_call(data, W, b2):
    return pl.pallas_call(
        _emb_body,
        grid=(N // _BLK,),
        in_specs=[
            pl.BlockSpec((_BLK, D), lambda i: (i, 0)),
            pl.BlockSpec((D, D), lambda i: (0, 0)),
            pl.BlockSpec((1, D), lambda i: (0, 0)),
        ],
        out_specs=pl.BlockSpec((_BLK, D), lambda i: (i, 0)),
        out_shape=jax.ShapeDtypeStruct((N, D), jnp.float32),
    )(data, W, b2)


def _upd_body(relu, out_ref, a0_ref, a1_ref, emb_ref, d0_ref, d1_ref, o_ref):
    deg = d0_ref[:, 0:1] + d1_ref[:, 0:1]
    pos = deg > 0.5
    invd = jnp.where(pos, (1.0 - TELEPORT) / jnp.where(pos, deg, 1.0), 0.0)
    a = jnp.where(pos, 1.0 - TELEPORT, 0.0)
    acc = a0_ref[...] + a1_ref[...]
    newo = a * out_ref[...] - invd * acc + TELEPORT * emb_ref[...]
    o_ref[...] = jnp.maximum(newo, 0.0) if relu else newo


def _upd_call(relu, out, acc0, acc1, emb, d0, d1):
    return pl.pallas_call(
        functools.partial(_upd_body, relu),
        grid=(N // _BLK,),
        in_specs=[
            pl.BlockSpec((_BLK, D), lambda i: (i, 0)),
            pl.BlockSpec((_BLK, D), lambda i: (i, 0)),
            pl.BlockSpec((_BLK, D), lambda i: (i, 0)),
            pl.BlockSpec((_BLK, D), lambda i: (i, 0)),
            pl.BlockSpec((_BLK, 16), lambda i: (i, 0)),
            pl.BlockSpec((_BLK, 16), lambda i: (i, 0)),
        ],
        out_specs=pl.BlockSpec((_BLK, D), lambda i: (i, 0)),
        out_shape=jax.ShapeDtypeStruct((N, D), jnp.float32),
    )(out, acc0, acc1, emb, d0, d1)


def kernel(data, edge_index, W, b):
    src = edge_index[0].astype(jnp.int32)
    dst = edge_index[1].astype(jnp.int32)
    pad = EPAD - E
    srcp = jnp.concatenate([src, jnp.zeros((pad,), jnp.int32)])
    dstp = jnp.concatenate([dst, jnp.full((pad,), N, jnp.int32)])

    emb = _emb_call(data, W, b.reshape(1, D))
    deg2 = _deg_step(dstp)
    d0, d1 = deg2[0], deg2[1]

    zer = jnp.zeros((C, D), jnp.float32)
    out = emb
    for t in range(DEPTH):
        acc2 = _edge_step(out, srcp, dstp, zer)
        out = _upd_call(t == DEPTH - 1, out, acc2[0], acc2[1], emb, d0, d1)
    return out
